# baseline (speedup 1.0000x reference)
import functools

import jax
import jax.numpy as jnp
from jax import lax
from jax.experimental import pallas as pl
from jax.experimental.pallas import tpu as pltpu
from jax.experimental.pallas import tpu_sc as plsc

B = 128
V = 100000
NC = 2
NS = 16
NW = NC * NS
S = 3136
VC = 448
NCHK = S // VC
NEG_INF = float("-inf")

_mesh = plsc.VectorSubcoreMesh(core_axis_name="c", subcore_axis_name="s")


@functools.partial(
    pl.kernel,
    out_type=[jax.ShapeDtypeStruct((NW * B,), jnp.int32),
              jax.ShapeDtypeStruct((NW * B,), jnp.float32)],
    mesh=_mesh,
    scratch_types=[
        pltpu.VMEM((2, VC, B), jnp.float32),
        pltpu.VMEM((16,), jnp.float32),
        pltpu.VMEM((16,), jnp.int32),
        pltpu.SemaphoreType.DMA,
        pltpu.SemaphoreType.DMA,
    ],
)
def _sc_scan(xt_hbm, outi_hbm, outv_hbm, buf, sv, si, sem0, sem1):
    cid = lax.axis_index("c")
    sid = lax.axis_index("s")
    wid = cid * NS + sid
    v0 = pl.multiple_of((wid * (V - S) // (NW - 1)) // 8 * 8, 8)
    sems = (sem0, sem1)
    lanes = lax.iota(jnp.int32, 16)
    zero_i = lanes * 0
    neginf_f = zero_i.astype(jnp.float32) + NEG_INF

    def start(k):
        return pltpu.async_copy(
            xt_hbm.at[pl.ds(v0 + k * VC, VC), :], buf.at[k % 2],
            sems[k % 2])

    bvs = [neginf_f for _ in range(8)]
    bis = [zero_i for _ in range(8)]

    descs = [None, None]
    descs[0] = start(0)
    for k in range(NCHK):
        if k + 1 < NCHK:
            descs[(k + 1) % 2] = start(k + 1)
        descs[k % 2].wait()
        bref = buf.at[k % 2]
        cbase = v0 + k * VC

        def it(v, carry):
            accs, civ = carry
            out = []
            for lg in range(8):
                bv, bi = accs[lg]
                x = bref[v, pl.ds(lg * 16, 16)]
                gt = x > bv
                bv = jnp.maximum(bv, x)
                bi = jnp.where(gt, civ, bi)
                out.append((bv, bi))
            return tuple(out), civ + 1

        civ0 = zero_i + cbase
        accs, _ = lax.fori_loop(
            0, VC, it,
            (tuple((bvs[lg], bis[lg]) for lg in range(8)), civ0),
            unroll=8)
        for lg in range(8):
            bvs[lg], bis[lg] = accs[lg]

    for lg in range(8):
        si[...] = bis[lg]
        pltpu.sync_copy(si, outi_hbm.at[pl.ds(wid * B + lg * 16, 16)])
        sv[...] = bvs[lg]
        pltpu.sync_copy(sv, outv_hbm.at[pl.ds(wid * B + lg * 16, 16)])


@functools.partial(
    pl.kernel,
    out_type=jax.ShapeDtypeStruct((B,), jnp.int32),
    mesh=_mesh,
    scratch_types=[
        pltpu.VMEM((NW * B,), jnp.float32),
        pltpu.VMEM((NW * B,), jnp.int32),
        pltpu.VMEM((16,), jnp.int32),
        pltpu.SemaphoreType.DMA,
        pltpu.SemaphoreType.DMA,
    ],
)
def _sc_merge(pi_hbm, pv_hbm, out_hbm, vbuf, ibuf, si, sem0, sem1):
    cid = lax.axis_index("c")
    sid = lax.axis_index("s")
    lanes = lax.iota(jnp.int32, 16)
    zero_i = lanes * 0
    neginf_f = zero_i.astype(jnp.float32) + NEG_INF

    @pl.when(sid % 4 == 0)
    def _():
        a = cid * 4 + sid // 4
        b0 = a * 16
        d0 = pltpu.async_copy(pv_hbm, vbuf, sem0)
        d1 = pltpu.async_copy(pi_hbm, ibuf, sem1)
        d0.wait()
        d1.wait()
        bv = neginf_f
        bi = zero_i
        for w in range(NW):
            v = vbuf[pl.ds(w * B + b0, 16)]
            i = ibuf[pl.ds(w * B + b0, 16)]
            gt = v > bv
            eq = v == bv
            bv = jnp.maximum(bv, v)
            bi = jnp.where(gt, i, bi)
            bi = jnp.where(eq, jnp.minimum(bi, i), bi)
        si[...] = bi
        pltpu.sync_copy(si, out_hbm.at[pl.ds(b0, 16)])


def kernel(m_logits):
    xt = m_logits.T
    pi, pv = _sc_scan(xt)
    out = _sc_merge(pi, pv)
    return out.reshape(B, 1).astype(jnp.int64)
